# baseline (speedup 1.0000x reference)
import functools

import jax
import jax.numpy as jnp
from jax import lax
from jax.experimental import pallas as pl
from jax.experimental.pallas import tpu as pltpu
from jax.experimental.pallas import tpu_sc as plsc

_VOCAB = 64
_PAIR = _VOCAB * _VOCAB
_LANES = 128
_ROWS_PER_CHUNK = 2
_CHUNK = _LANES * _ROWS_PER_CHUNK
_NBUF = 2


def _table_body(e_ref, w_ref, b_ref, t_ref):
    t_ref[...] = lax.dot_general(
        e_ref[...], w_ref[...], (((1,), (1,)), ((), ())),
        preferred_element_type=jnp.float32) + b_ref[...]


def _pair_table_body(t_ref, out_ref):
    a = pl.program_id(0)
    left = jnp.broadcast_to(t_ref[pl.ds(a, 1), :], (_VOCAB, _VOCAB))
    out_ref[...] = jnp.concatenate([left, t_ref[...]], axis=1)


def _fuse_pair_table(embed_table, W, b):
    t = pl.pallas_call(
        _table_body,
        out_shape=jax.ShapeDtypeStruct((_VOCAB, _VOCAB), jnp.float32),
    )(embed_table, W, b.reshape(1, _VOCAB))
    return pl.pallas_call(
        _pair_table_body,
        grid=(_VOCAB,),
        in_specs=[pl.BlockSpec((_VOCAB, _VOCAB), lambda a: (0, 0))],
        out_specs=pl.BlockSpec((_VOCAB, 2 * _VOCAB), lambda a: (a, 0)),
        out_shape=jax.ShapeDtypeStruct((_PAIR, 2 * _VOCAB), jnp.float32),
    )(t)


def _pair_idx_body(e_ref, o_ref, out_ref):
    out_ref[...] = e_ref[...] * _VOCAB + o_ref[...]


def _pair_idx(ids_even, ids_odd):
    n = ids_even.shape[0]
    return pl.pallas_call(
        _pair_idx_body,
        out_shape=jax.ShapeDtypeStruct((n, _LANES), jnp.int32),
    )(ids_even, ids_odd)


@functools.lru_cache(maxsize=None)
def _gather_nw():
    info = plsc.get_sparse_core_info()
    return info.num_cores * info.num_subcores


@functools.lru_cache(maxsize=None)
def _make_gather(rows_total):
    info = plsc.get_sparse_core_info()
    nc, ns = info.num_cores, info.num_subcores
    nw = nc * ns
    rows_per_w = rows_total // nw
    n_chunks = rows_per_w // _ROWS_PER_CHUNK

    mesh = plsc.VectorSubcoreMesh(core_axis_name="c", subcore_axis_name="s")

    @functools.partial(
        pl.kernel, mesh=mesh,
        out_type=jax.ShapeDtypeStruct((rows_total * _LANES, 2 * _VOCAB),
                                      jnp.float32),
        scratch_types=[
            pltpu.VMEM_SHARED((_PAIR, 2 * _VOCAB), jnp.float32),
            pltpu.VMEM((rows_per_w, _LANES), jnp.int32),
            pltpu.VMEM((_NBUF, _CHUNK, 2 * _VOCAB), jnp.float32),
            pltpu.SemaphoreType.DMA((_NBUF,)),
            pltpu.SemaphoreType.DMA((_NBUF,)),
        ],
    )
    def gather(table_hbm, idx_hbm, out_hbm, table_sh, idx_v, rows_v,
               gsem, wsem):
        cid = lax.axis_index("c")
        sid = lax.axis_index("s")
        wid = sid * nc + cid
        row0 = wid * rows_per_w
        out0 = row0 * _LANES

        @pl.when(sid == 0)
        def _():
            pltpu.sync_copy(table_hbm, table_sh)
        plsc.subcore_barrier()

        pltpu.sync_copy(idx_hbm.at[wid], idx_v)

        def fire_gather(chunk, buf):
            for j in range(_ROWS_PER_CHUNK):
                pltpu.async_copy(
                    table_sh.at[idx_v.at[chunk * _ROWS_PER_CHUNK + j]],
                    rows_v.at[buf].at[pl.ds(j * _LANES, _LANES)],
                    gsem.at[buf])

        def wait_gather(buf):
            pltpu.make_async_copy(
                out_hbm.at[pl.ds(0, _CHUNK)], rows_v.at[buf],
                gsem.at[buf]).wait()

        def fire_wb(chunk, buf):
            pltpu.async_copy(
                rows_v.at[buf],
                out_hbm.at[pl.ds(out0 + chunk * _CHUNK, _CHUNK)],
                wsem.at[buf])

        def wait_wb(buf):
            pltpu.make_async_copy(
                rows_v.at[buf], out_hbm.at[pl.ds(0, _CHUNK)],
                wsem.at[buf]).wait()

        for b in range(_NBUF):
            fire_gather(b, b)

        def body(k, _):
            buf = lax.rem(k, _NBUF)
            wait_gather(buf)
            fire_wb(k, buf)

            @pl.when(k + _NBUF < n_chunks)
            def _():
                wait_wb(buf)
                fire_gather(k + _NBUF, buf)

            return 0

        lax.fori_loop(0, n_chunks, body, 0)

        for b in range(_NBUF):
            wait_wb(b)

    return gather


_MBLK = 4


def _xpose_body(f_ref, out_ref):
    eye = jnp.eye(_LANES, dtype=jnp.float32)
    for j in range(_MBLK):
        blk = f_ref[:, j, 0, :]
        t = lax.dot_general(blk, eye, (((0,), (0,)), ((), ())),
                            preferred_element_type=jnp.float32)
        out_ref[2 * j:2 * j + 2, :, :] = t.reshape(2, _VOCAB, _LANES)


def _xpose(flat, bsz, seq):
    m_tot = seq // 2
    f4 = flat.reshape(bsz, m_tot, 1, _LANES)
    grid = (m_tot // _MBLK, bsz // _LANES)
    out_phys = pl.pallas_call(
        _xpose_body,
        grid=grid,
        in_specs=[pl.BlockSpec((_LANES, _MBLK, 1, _LANES),
                               lambda m, bb: (bb, m, 0, 0))],
        out_specs=pl.BlockSpec((2 * _MBLK, _VOCAB, _LANES),
                               lambda m, bb: (m, 0, bb)),
        out_shape=jax.ShapeDtypeStruct((seq, _VOCAB, bsz), jnp.float32),
    )(f4)
    return jnp.transpose(out_phys, (2, 0, 1))


def kernel(input_ids, embed_table, W, b):
    table2 = _fuse_pair_table(embed_table, W, b)
    bsz, seq = input_ids.shape
    n_pairs = bsz * seq // 2
    rows_total = n_pairs // _LANES
    ids = input_ids.reshape(n_pairs, 2).astype(jnp.int32)
    ids_even = ids[:, 0].reshape(rows_total, _LANES)
    ids_odd = ids[:, 1].reshape(rows_total, _LANES)
    idx = _pair_idx(ids_even, ids_odd)
    nw = _gather_nw()
    idx = idx.reshape(nw, rows_total // nw, _LANES)
    flat = _make_gather(rows_total)(table2, idx)
    return _xpose(flat, bsz, seq)


# speedup vs baseline: 2.6509x; 2.6509x over previous
import functools

import jax
import jax.numpy as jnp
from jax import lax
from jax.experimental import pallas as pl
from jax.experimental.pallas import tpu as pltpu
from jax.experimental.pallas import tpu_sc as plsc

_VOCAB = 64
_PAIR = _VOCAB * _VOCAB
_LANES = 128
_ROWS_PER_CHUNK = 2
_CHUNK = _LANES * _ROWS_PER_CHUNK
_NBUF = 2


def _table_body(e_ref, w_ref, b_ref, t_ref):
    t_ref[...] = lax.dot_general(
        e_ref[...], w_ref[...], (((1,), (1,)), ((), ())),
        preferred_element_type=jnp.float32) + b_ref[...]


def _pair_table_body(t_ref, out_ref):
    a = pl.program_id(0)
    left = jnp.broadcast_to(t_ref[pl.ds(a, 1), :], (_VOCAB, _VOCAB))
    out_ref[...] = jnp.concatenate([left, t_ref[...]], axis=1)


def _fuse_pair_table(embed_table, W, b):
    t = pl.pallas_call(
        _table_body,
        out_shape=jax.ShapeDtypeStruct((_VOCAB, _VOCAB), jnp.float32),
    )(embed_table, W, b.reshape(1, _VOCAB))
    return pl.pallas_call(
        _pair_table_body,
        grid=(_VOCAB,),
        in_specs=[pl.BlockSpec((_VOCAB, _VOCAB), lambda a: (0, 0))],
        out_specs=pl.BlockSpec((_VOCAB, 2 * _VOCAB), lambda a: (a, 0)),
        out_shape=jax.ShapeDtypeStruct((_PAIR, 2 * _VOCAB), jnp.float32),
    )(t)


def _pair_idx_body(e_ref, o_ref, out_ref):
    out_ref[...] = e_ref[...] * _VOCAB + o_ref[...]


def _pair_idx(ids_even, ids_odd):
    n = ids_even.shape[0]
    return pl.pallas_call(
        _pair_idx_body,
        out_shape=jax.ShapeDtypeStruct((n, _LANES), jnp.int32),
    )(ids_even, ids_odd)


@functools.lru_cache(maxsize=None)
def _gather_nw():
    info = plsc.get_sparse_core_info()
    return info.num_cores * info.num_subcores


@functools.lru_cache(maxsize=None)
def _make_gather(rows_total):
    info = plsc.get_sparse_core_info()
    nc, ns = info.num_cores, info.num_subcores
    nw = nc * ns
    rows_per_w = rows_total // nw
    n_chunks = rows_per_w // _ROWS_PER_CHUNK

    mesh = plsc.VectorSubcoreMesh(core_axis_name="c", subcore_axis_name="s")

    @functools.partial(
        pl.kernel, mesh=mesh,
        out_type=jax.ShapeDtypeStruct((rows_total * _LANES, 2 * _VOCAB),
                                      jnp.float32),
        scratch_types=[
            pltpu.VMEM_SHARED((_PAIR, 2 * _VOCAB), jnp.float32),
            pltpu.VMEM((rows_per_w, _LANES), jnp.int32),
            pltpu.VMEM((_NBUF, _CHUNK, 2 * _VOCAB), jnp.float32),
            pltpu.SemaphoreType.DMA((_NBUF,)),
            pltpu.SemaphoreType.DMA((_NBUF,)),
        ],
    )
    def gather(table_hbm, idx_hbm, out_hbm, table_sh, idx_v, rows_v,
               gsem, wsem):
        cid = lax.axis_index("c")
        sid = lax.axis_index("s")
        wid = sid * nc + cid
        row0 = wid * rows_per_w
        out0 = row0 * _LANES

        @pl.when(sid == 0)
        def _():
            pltpu.sync_copy(table_hbm, table_sh)
        plsc.subcore_barrier()

        pltpu.sync_copy(idx_hbm.at[wid], idx_v)

        def fire_gather(chunk, buf):
            for j in range(_ROWS_PER_CHUNK):
                pltpu.async_copy(
                    table_sh.at[idx_v.at[chunk * _ROWS_PER_CHUNK + j]],
                    rows_v.at[buf].at[pl.ds(j * _LANES, _LANES)],
                    gsem.at[buf])

        def wait_gather(buf):
            pltpu.make_async_copy(
                out_hbm.at[pl.ds(0, _CHUNK)], rows_v.at[buf],
                gsem.at[buf]).wait()

        def fire_wb(chunk, buf):
            pltpu.async_copy(
                rows_v.at[buf],
                out_hbm.at[pl.ds(out0 + chunk * _CHUNK, _CHUNK)],
                wsem.at[buf])

        def wait_wb(buf):
            pltpu.make_async_copy(
                rows_v.at[buf], out_hbm.at[pl.ds(0, _CHUNK)],
                wsem.at[buf]).wait()

        for b in range(_NBUF):
            fire_gather(b, b)

        def body(k, _):
            buf = lax.rem(k, _NBUF)
            wait_gather(buf)
            fire_wb(k, buf)

            @pl.when(k + _NBUF < n_chunks)
            def _():
                wait_wb(buf)
                fire_gather(k + _NBUF, buf)

            return 0

        lax.fori_loop(0, n_chunks, body, 0)

        for b in range(_NBUF):
            wait_wb(b)

    return gather


def _xpose_body(f_ref, out_ref):
    nb = f_ref.shape[1] // _LANES
    eye = jnp.eye(_LANES, dtype=jnp.float32)
    for k in range(nb):
        blk = f_ref[0, pl.ds(k * _LANES, _LANES), :]
        t = lax.dot_general(blk, eye, (((0,), (0,)), ((), ())),
                            precision=lax.Precision.HIGHEST,
                            preferred_element_type=jnp.float32)
        out_ref[:, :, pl.ds(k * _LANES, _LANES)] = t.reshape(2, _VOCAB,
                                                             _LANES)


def _xpose(flat, bsz, seq):
    m_tot = seq // 2
    f3 = flat.reshape(m_tot, bsz, _LANES)
    out_phys = pl.pallas_call(
        _xpose_body,
        grid=(m_tot,),
        in_specs=[pl.BlockSpec((1, bsz, _LANES), lambda m: (m, 0, 0))],
        out_specs=pl.BlockSpec((2, _VOCAB, bsz), lambda m: (m, 0, 0)),
        out_shape=jax.ShapeDtypeStruct((seq, _VOCAB, bsz), jnp.float32),
    )(f3)
    return jnp.transpose(out_phys, (2, 0, 1))


def kernel(input_ids, embed_table, W, b):
    table2 = _fuse_pair_table(embed_table, W, b)
    bsz, seq = input_ids.shape
    n_pairs = bsz * seq // 2
    rows_total = n_pairs // _LANES
    m_tot = seq // 2
    ids = input_ids.reshape(bsz, m_tot, 2).astype(jnp.int32)
    ids_even = ids[:, :, 0].T.reshape(rows_total, _LANES)
    ids_odd = ids[:, :, 1].T.reshape(rows_total, _LANES)
    idx = _pair_idx(ids_even, ids_odd)
    nw = _gather_nw()
    idx = idx.reshape(nw, rows_total // nw, _LANES)
    flat = _make_gather(rows_total)(table2, idx)
    return _xpose(flat, bsz, seq)


# speedup vs baseline: 3.2164x; 1.2133x over previous
import functools

import jax
import jax.numpy as jnp
from jax import lax
from jax.experimental import pallas as pl
from jax.experimental.pallas import tpu as pltpu
from jax.experimental.pallas import tpu_sc as plsc

_VOCAB = 64
_PAIR = _VOCAB * _VOCAB
_LANES = 128
_ROWS_PER_CHUNK = 2
_CHUNK = _LANES * _ROWS_PER_CHUNK
_NBUF = 2


def _table_body(e_ref, w_ref, b_ref, t_ref):
    t_ref[...] = lax.dot_general(
        e_ref[...], w_ref[...], (((1,), (1,)), ((), ())),
        preferred_element_type=jnp.float32) + b_ref[...]


def _pair_table_body(t_ref, out_ref):
    a = pl.program_id(0)
    left = jnp.broadcast_to(t_ref[pl.ds(a, 1), :], (_VOCAB, _VOCAB))
    out_ref[...] = jnp.concatenate([left, t_ref[...]], axis=1)


def _fuse_pair_table(embed_table, W, b):
    t = pl.pallas_call(
        _table_body,
        out_shape=jax.ShapeDtypeStruct((_VOCAB, _VOCAB), jnp.float32),
    )(embed_table, W, b.reshape(1, _VOCAB))
    return pl.pallas_call(
        _pair_table_body,
        grid=(_VOCAB,),
        in_specs=[pl.BlockSpec((_VOCAB, _VOCAB), lambda a: (0, 0))],
        out_specs=pl.BlockSpec((_VOCAB, 2 * _VOCAB), lambda a: (a, 0)),
        out_shape=jax.ShapeDtypeStruct((_PAIR, 2 * _VOCAB), jnp.float32),
    )(t)


def _pair_idx_body(e_ref, o_ref, out_ref):
    out_ref[...] = e_ref[...] * _VOCAB + o_ref[...]


def _pair_idx(ids_even, ids_odd):
    n = ids_even.shape[0]
    return pl.pallas_call(
        _pair_idx_body,
        out_shape=jax.ShapeDtypeStruct((n, _LANES), jnp.int32),
    )(ids_even, ids_odd)


@functools.lru_cache(maxsize=None)
def _gather_nw():
    info = plsc.get_sparse_core_info()
    return info.num_cores * info.num_subcores


@functools.lru_cache(maxsize=None)
def _make_gather(rows_total):
    info = plsc.get_sparse_core_info()
    nc, ns = info.num_cores, info.num_subcores
    nw = nc * ns
    rows_per_w = rows_total // nw
    n_chunks = rows_per_w // _ROWS_PER_CHUNK

    mesh = plsc.VectorSubcoreMesh(core_axis_name="c", subcore_axis_name="s")

    @functools.partial(
        pl.kernel, mesh=mesh,
        out_type=jax.ShapeDtypeStruct((rows_total * _LANES, 2 * _VOCAB),
                                      jnp.float32),
        scratch_types=[
            pltpu.VMEM_SHARED((_PAIR, 2 * _VOCAB), jnp.float32),
            pltpu.VMEM((rows_per_w, _LANES), jnp.int32),
            pltpu.VMEM((_NBUF, _CHUNK, 2 * _VOCAB), jnp.float32),
            pltpu.SemaphoreType.DMA((_NBUF,)),
            pltpu.SemaphoreType.DMA((_NBUF,)),
        ],
    )
    def gather(table_hbm, idx_hbm, out_hbm, table_sh, idx_v, rows_v,
               gsem, wsem):
        cid = lax.axis_index("c")
        sid = lax.axis_index("s")
        wid = sid * nc + cid
        row0 = wid * rows_per_w
        out0 = row0 * _LANES

        @pl.when(sid == 0)
        def _():
            pltpu.sync_copy(table_hbm, table_sh)
        plsc.subcore_barrier()

        pltpu.sync_copy(idx_hbm.at[wid], idx_v)

        def fire_gather(chunk, buf):
            for j in range(_ROWS_PER_CHUNK):
                pltpu.async_copy(
                    table_sh.at[idx_v.at[chunk * _ROWS_PER_CHUNK + j]],
                    rows_v.at[buf].at[pl.ds(j * _LANES, _LANES)],
                    gsem.at[buf])

        def wait_gather(buf):
            pltpu.make_async_copy(
                out_hbm.at[pl.ds(0, _CHUNK)], rows_v.at[buf],
                gsem.at[buf]).wait()

        def fire_wb(chunk, buf):
            pltpu.async_copy(
                rows_v.at[buf],
                out_hbm.at[pl.ds(out0 + chunk * _CHUNK, _CHUNK)],
                wsem.at[buf])

        def wait_wb(buf):
            pltpu.make_async_copy(
                rows_v.at[buf], out_hbm.at[pl.ds(0, _CHUNK)],
                wsem.at[buf]).wait()

        for b in range(_NBUF):
            fire_gather(b, b)

        def body(k, _):
            buf = lax.rem(k, _NBUF)
            wait_gather(buf)
            fire_wb(k, buf)

            @pl.when(k + _NBUF < n_chunks)
            def _():
                wait_wb(buf)
                fire_gather(k + _NBUF, buf)

            return 0

        lax.fori_loop(0, n_chunks, body, 0)

        for b in range(_NBUF):
            wait_wb(b)

    return gather


def _xpose_body(f_ref, out_ref):
    nb = f_ref.shape[1] // _LANES
    for k in range(nb):
        blk = f_ref[0, pl.ds(k * _LANES, _LANES), :]
        t = blk.T
        out_ref[:, :, pl.ds(k * _LANES, _LANES)] = t.reshape(2, _VOCAB,
                                                             _LANES)


def _xpose(flat, bsz, seq):
    m_tot = seq // 2
    f3 = flat.reshape(m_tot, bsz, _LANES)
    out_phys = pl.pallas_call(
        _xpose_body,
        grid=(m_tot,),
        in_specs=[pl.BlockSpec((1, bsz, _LANES), lambda m: (m, 0, 0))],
        out_specs=pl.BlockSpec((2, _VOCAB, bsz), lambda m: (m, 0, 0)),
        out_shape=jax.ShapeDtypeStruct((seq, _VOCAB, bsz), jnp.float32),
    )(f3)
    return jnp.transpose(out_phys, (2, 0, 1))


def kernel(input_ids, embed_table, W, b):
    table2 = _fuse_pair_table(embed_table, W, b)
    bsz, seq = input_ids.shape
    n_pairs = bsz * seq // 2
    rows_total = n_pairs // _LANES
    m_tot = seq // 2
    ids = input_ids.reshape(bsz, m_tot, 2).astype(jnp.int32)
    ids_even = ids[:, :, 0].T.reshape(rows_total, _LANES)
    ids_odd = ids[:, :, 1].T.reshape(rows_total, _LANES)
    idx = _pair_idx(ids_even, ids_odd)
    nw = _gather_nw()
    idx = idx.reshape(nw, rows_total // nw, _LANES)
    flat = _make_gather(rows_total)(table2, idx)
    return _xpose(flat, bsz, seq)


# speedup vs baseline: 3.2839x; 1.0210x over previous
import functools

import jax
import jax.numpy as jnp
from jax import lax
from jax.experimental import pallas as pl
from jax.experimental.pallas import tpu as pltpu
from jax.experimental.pallas import tpu_sc as plsc

_VOCAB = 64
_PAIR = _VOCAB * _VOCAB
_LANES = 128
_ROWS_PER_CHUNK = 2
_CHUNK = _LANES * _ROWS_PER_CHUNK
_NBUF = 2


def _table_body(e_ref, w_ref, b_ref, t_ref):
    t_ref[...] = lax.dot_general(
        e_ref[...], w_ref[...], (((1,), (1,)), ((), ())),
        preferred_element_type=jnp.float32) + b_ref[...]


def _pair_table_body(t_ref, out_ref):
    a = pl.program_id(0)
    left = jnp.broadcast_to(t_ref[pl.ds(a, 1), :], (_VOCAB, _VOCAB))
    out_ref[...] = jnp.concatenate([left, t_ref[...]], axis=1)


def _fuse_pair_table(embed_table, W, b):
    t = pl.pallas_call(
        _table_body,
        out_shape=jax.ShapeDtypeStruct((_VOCAB, _VOCAB), jnp.float32),
    )(embed_table, W, b.reshape(1, _VOCAB))
    return pl.pallas_call(
        _pair_table_body,
        grid=(_VOCAB,),
        in_specs=[pl.BlockSpec((_VOCAB, _VOCAB), lambda a: (0, 0))],
        out_specs=pl.BlockSpec((_VOCAB, 2 * _VOCAB), lambda a: (a, 0)),
        out_shape=jax.ShapeDtypeStruct((_PAIR, 2 * _VOCAB), jnp.float32),
    )(t)


def _pair_idx_body(e_ref, o_ref, out_ref):
    out_ref[...] = e_ref[...] * _VOCAB + o_ref[...]


def _pair_idx(ids_even, ids_odd):
    n = ids_even.shape[0]
    return pl.pallas_call(
        _pair_idx_body,
        out_shape=jax.ShapeDtypeStruct((n, _LANES), jnp.int32),
    )(ids_even, ids_odd)


@functools.lru_cache(maxsize=None)
def _gather_nw():
    info = plsc.get_sparse_core_info()
    return info.num_cores * info.num_subcores


@functools.lru_cache(maxsize=None)
def _make_gather(rows_total):
    info = plsc.get_sparse_core_info()
    nc, ns = info.num_cores, info.num_subcores
    nw = nc * ns
    rows_per_w = rows_total // nw
    n_chunks = rows_per_w // _ROWS_PER_CHUNK

    mesh = plsc.VectorSubcoreMesh(core_axis_name="c", subcore_axis_name="s")

    @functools.partial(
        pl.kernel, mesh=mesh,
        out_type=jax.ShapeDtypeStruct((rows_total * _LANES, 2 * _VOCAB),
                                      jnp.float32),
        scratch_types=[
            pltpu.VMEM_SHARED((_PAIR, 2 * _VOCAB), jnp.float32),
            pltpu.VMEM((rows_per_w, _LANES), jnp.int32),
            pltpu.VMEM((_NBUF, _CHUNK, 2 * _VOCAB), jnp.float32),
            pltpu.SemaphoreType.DMA((_NBUF,)),
            pltpu.SemaphoreType.DMA((_NBUF,)),
        ],
    )
    def gather(table_hbm, idx_hbm, out_hbm, table_sh, idx_v, rows_v,
               gsem, wsem):
        cid = lax.axis_index("c")
        sid = lax.axis_index("s")
        wid = sid * nc + cid
        row0 = wid * rows_per_w
        out0 = row0 * _LANES

        @pl.when(sid == 0)
        def _():
            pltpu.sync_copy(table_hbm, table_sh)
        plsc.subcore_barrier()

        pltpu.sync_copy(idx_hbm.at[wid], idx_v)

        def fire_gather(chunk, buf):
            for j in range(_ROWS_PER_CHUNK):
                pltpu.async_copy(
                    table_sh.at[idx_v.at[chunk * _ROWS_PER_CHUNK + j]],
                    rows_v.at[buf].at[pl.ds(j * _LANES, _LANES)],
                    gsem.at[buf])

        def wait_gather(buf):
            pltpu.make_async_copy(
                out_hbm.at[pl.ds(0, _CHUNK)], rows_v.at[buf],
                gsem.at[buf]).wait()

        def fire_wb(chunk, buf):
            pltpu.async_copy(
                rows_v.at[buf],
                out_hbm.at[pl.ds(out0 + chunk * _CHUNK, _CHUNK)],
                wsem.at[buf])

        def wait_wb(buf):
            pltpu.make_async_copy(
                rows_v.at[buf], out_hbm.at[pl.ds(0, _CHUNK)],
                wsem.at[buf]).wait()

        for b in range(_NBUF):
            fire_gather(b, b)

        def body(k, _):
            buf = lax.rem(k, _NBUF)
            wait_gather(buf)
            fire_wb(k, buf)

            @pl.when(k + _NBUF < n_chunks)
            def _():
                wait_wb(buf)
                fire_gather(k + _NBUF, buf)

            return 0

        lax.fori_loop(0, n_chunks, body, 0)

        for b in range(_NBUF):
            wait_wb(b)

    return gather


def _xpose_body(f_ref, out_ref):
    nb = f_ref.shape[1] // _LANES
    for k in range(nb):
        blk = f_ref[0, pl.ds(k * _LANES, _LANES), :]
        t = blk.T
        out_ref[:, :, pl.ds(k * _LANES, _LANES)] = t.reshape(2, _VOCAB,
                                                             _LANES)


def _xpose_body_chained(f_ref, prev_ref, out_ref):
    del prev_ref
    _xpose_body(f_ref, out_ref)


def _out_map(off, m):
    return (off + m, 0, 0)


def _xpose_parts(parts, bsz, seq):
    m_each = (seq // 2) // len(parts)
    out_shape = jax.ShapeDtypeStruct((seq, _VOCAB, bsz), jnp.float32)
    acc = None
    for i, flat_h in enumerate(parts):
        f3 = flat_h.reshape(m_each, bsz, _LANES)
        in_specs = [pl.BlockSpec((1, bsz, _LANES), lambda m: (m, 0, 0))]
        ops = [f3]
        kwargs = {}
        body = _xpose_body
        if acc is not None:
            in_specs.append(pl.BlockSpec(memory_space=pl.ANY))
            ops.append(acc)
            kwargs = dict(input_output_aliases={1: 0})
            body = _xpose_body_chained
        acc = pl.pallas_call(
            body,
            grid=(m_each,),
            in_specs=in_specs,
            out_specs=pl.BlockSpec((2, _VOCAB, bsz),
                                   functools.partial(_out_map, i * m_each)),
            out_shape=out_shape,
            **kwargs)(*ops)
    return jnp.transpose(acc, (2, 0, 1))


def kernel(input_ids, embed_table, W, b):
    table2 = _fuse_pair_table(embed_table, W, b)
    bsz, seq = input_ids.shape
    n_pairs = bsz * seq // 2
    rows_total = n_pairs // _LANES
    m_tot = seq // 2
    ids = input_ids.reshape(bsz, m_tot, 2).astype(jnp.int32)
    ids_even = ids[:, :, 0].T.reshape(rows_total, _LANES)
    ids_odd = ids[:, :, 1].T.reshape(rows_total, _LANES)
    idx = _pair_idx(ids_even, ids_odd)
    nw = _gather_nw()
    n_parts = 2
    rows_part = rows_total // n_parts
    gather = _make_gather(rows_part)
    parts = []
    for i in range(n_parts):
        idx_h = idx[i * rows_part:(i + 1) * rows_part]
        idx_h = idx_h.reshape(nw, rows_part // nw, _LANES)
        parts.append(gather(table2, idx_h))
    return _xpose_parts(parts, bsz, seq)
